# baseline (speedup 1.0000x reference)
import functools
import math

import jax
import jax.numpy as jnp
from jax import lax
from jax.experimental import pallas as pl
from jax.experimental.pallas import tpu as pltpu
from jax.experimental.pallas import tpu_sc as plsc

_B, _S, _D = 2, 2048, 768
_E, _L, _TOPK = 5, 2, 2
_RK = 512
_RA = 256
_PAD = 16


def _gelu(x):
    return 0.5 * x * (1.0 + lax.erf(x * (1.0 / math.sqrt(2.0))))


def _attn_body(hid_ref, hidb_ref, wq_ref, bq_ref, wk_ref, bk_ref,
               pooled_ref, swh_ref, k_scr, cs_scr):
    i = pl.program_id(1)
    nblk = _S // _RA

    @pl.when(i == 0)
    def _init():
        k_scr[...] = (
            jnp.dot(hidb_ref[0], wk_ref[...],
                    preferred_element_type=jnp.float32)
            + bk_ref[...]
        ).astype(jnp.bfloat16)
        pooled_ref[0] = jnp.sum(hid_ref[0], axis=0, keepdims=True) * (1.0 / _S)
        cs_scr[...] = jnp.zeros((1, _S), jnp.float32)

    @pl.when(i > 0)
    def _block():
        r0 = (i - 1) * _RA
        q = (
            jnp.dot(hidb_ref[0, pl.ds(r0, _RA), :], wq_ref[...],
                    preferred_element_type=jnp.float32)
            + bq_ref[...]
        ).astype(jnp.bfloat16)
        p = jnp.exp(lax.dot_general(
            q, k_scr[...], (((1,), (1,)), ((), ())),
            preferred_element_type=jnp.float32,
        ) * (1.0 / math.sqrt(_D)))
        rw = ((1.0 / _S) / jnp.sum(p, axis=1, keepdims=True)
              ).astype(jnp.bfloat16)
        cs_scr[...] += lax.dot_general(
            rw, p.astype(jnp.bfloat16), (((0,), (0,)), ((), ())),
            preferred_element_type=jnp.float32)

    @pl.when(i == nblk)
    def _fin():
        swh_ref[0] = jnp.dot(cs_scr[...], hid_ref[0],
                             preferred_element_type=jnp.float32)


def _attn(hs, hsb, Wqb, bq2, Wkb, bk2):
    return pl.pallas_call(
        _attn_body,
        grid=(_B, 1 + _S // _RA),
        in_specs=[
            pl.BlockSpec((1, _S, _D), lambda b, i: (b, 0, 0)),
            pl.BlockSpec((1, _S, _D), lambda b, i: (b, 0, 0)),
            pl.BlockSpec((_D, _D), lambda b, i: (0, 0)),
            pl.BlockSpec((1, _D), lambda b, i: (0, 0)),
            pl.BlockSpec((_D, _D), lambda b, i: (0, 0)),
            pl.BlockSpec((1, _D), lambda b, i: (0, 0)),
        ],
        out_specs=[
            pl.BlockSpec((1, 1, _D), lambda b, i: (b, 0, 0)),
            pl.BlockSpec((1, 1, _D), lambda b, i: (b, 0, 0)),
        ],
        out_shape=[
            jax.ShapeDtypeStruct((_B, 1, _D), jnp.float32),
            jax.ShapeDtypeStruct((_B, 1, _D), jnp.float32),
        ],
        scratch_shapes=[pltpu.VMEM((_S, _D), jnp.bfloat16),
                        pltpu.VMEM((1, _S), jnp.float32)],
        compiler_params=pltpu.CompilerParams(
            dimension_semantics=("arbitrary", "arbitrary")),
    )(hs, hsb, Wqb, bq2, Wkb, bk2)


def _head_body(pooled_ref, swh_ref, wv_ref, bv_ref, wea_ref, bea_ref,
               wg1_ref, bg1_ref, wg2_ref, bg2_ref,
               we1_ref, be1_ref, we2_ref, be2_ref,
               probs_ref, eo_ref):
    e = pl.program_id(0)

    @pl.when(e == 0)
    def _gate():
        seq = (
            jnp.dot(swh_ref[:, 0, :], wv_ref[...],
                    preferred_element_type=jnp.float32)
            + bv_ref[...]
        )
        ea = jnp.dot(seq, wea_ref[...],
                     preferred_element_type=jnp.float32) + bea_ref[...]
        g1 = _gelu(jnp.dot(seq, wg1_ref[...],
                           preferred_element_type=jnp.float32) + bg1_ref[...])
        logits = ea + jnp.dot(g1, wg2_ref[...],
                              preferred_element_type=jnp.float32) + bg2_ref[...]
        mx = jnp.max(logits, axis=1, keepdims=True)
        ex = jnp.exp(logits - mx)
        probs = ex / jnp.sum(ex, axis=1, keepdims=True)
        probs_ref[:, 0:_E] = probs
        probs_ref[:, _E:] = jnp.zeros((_B, _PAD - _E), jnp.float32)

    h1 = _gelu(
        jnp.dot(pooled_ref[:, 0, :], we1_ref[0],
                preferred_element_type=jnp.float32)
        + be1_ref[0]
    )
    eo_ref[0] = jnp.dot(h1, we2_ref[0],
                        preferred_element_type=jnp.float32) + be2_ref[0]


def _head(pooled, swh, Wv, bv2, W_ea, bea2, Wg1, bg12, Wg2, bg22,
          We1, be1_3, We2, be2_3):
    return pl.pallas_call(
        _head_body,
        grid=(_E,),
        in_specs=[
            pl.BlockSpec((_B, 1, _D), lambda e: (0, 0, 0)),
            pl.BlockSpec((_B, 1, _D), lambda e: (0, 0, 0)),
            pl.BlockSpec((_D, _D), lambda e: (0, 0)),
            pl.BlockSpec((1, _D), lambda e: (0, 0)),
            pl.BlockSpec((_D, _E), lambda e: (0, 0)),
            pl.BlockSpec((1, _E), lambda e: (0, 0)),
            pl.BlockSpec((_D, _D), lambda e: (0, 0)),
            pl.BlockSpec((1, _D), lambda e: (0, 0)),
            pl.BlockSpec((_D, _E), lambda e: (0, 0)),
            pl.BlockSpec((1, _E), lambda e: (0, 0)),
            pl.BlockSpec((1, _D, _D), lambda e: (e, 0, 0)),
            pl.BlockSpec((1, 1, _D), lambda e: (e, 0, 0)),
            pl.BlockSpec((1, _D, _L), lambda e: (e, 0, 0)),
            pl.BlockSpec((1, 1, _L), lambda e: (e, 0, 0)),
        ],
        out_specs=[
            pl.BlockSpec((_B, _PAD), lambda e: (0, 0)),
            pl.BlockSpec((1, _B, _L), lambda e: (e, 0, 0)),
        ],
        out_shape=[
            jax.ShapeDtypeStruct((_B, _PAD), jnp.float32),
            jax.ShapeDtypeStruct((_E, _B, _L), jnp.float32),
        ],
        compiler_params=pltpu.CompilerParams(
            dimension_semantics=("arbitrary",)),
    )(pooled, swh, Wv, bv2, W_ea, bea2, Wg1, bg12, Wg2, bg22,
      We1, be1_3, We2, be2_3)


def _route_sc(probs_pad, eo_pad):
    mesh = plsc.VectorSubcoreMesh(core_axis_name="c", subcore_axis_name="s")

    @functools.partial(
        pl.kernel,
        mesh=mesh,
        out_type=[
            jax.ShapeDtypeStruct((_B, _PAD), jnp.float32),
            jax.ShapeDtypeStruct((_B, _PAD), jnp.int32),
        ],
        scratch_types=[
            pltpu.VMEM((_PAD,), jnp.float32),
            pltpu.VMEM((_PAD,), jnp.float32),
            pltpu.VMEM((_PAD,), jnp.float32),
            pltpu.VMEM((_PAD,), jnp.int32),
        ],
    )
    def run(probs_hbm, eo_hbm, oval_hbm, oidx_hbm,
            probs_v, eo_v, oval_v, oidx_v):
        wid = lax.axis_index("s") * 2 + lax.axis_index("c")

        @pl.when(wid < _B)
        def _():
            b = wid
            pltpu.sync_copy(probs_hbm.at[b], probs_v)
            pltpu.sync_copy(eo_hbm.at[b], eo_v)
            p = probs_v[...]
            lane = lax.iota(jnp.int32, _PAD)

            def _shuf(x, i):
                dnums = lax.GatherDimensionNumbers(
                    offset_dims=(), collapsed_slice_dims=(0,),
                    start_index_map=(0,))
                return lax.gather(
                    x, i[:, None], dnums, (1,),
                    mode=lax.GatherScatterMode.PROMISE_IN_BOUNDS)

            def _allmax(v):
                for sh in (1, 2, 4, 8):
                    v = jnp.maximum(v, _shuf(v, lane ^ sh))
                return v

            def _allmin(v):
                for sh in (1, 2, 4, 8):
                    v = jnp.minimum(v, _shuf(v, lane ^ sh))
                return v

            m1 = _allmax(p)
            i1 = _allmin(jnp.where(p == m1, lane, _PAD))
            p2 = jnp.where(lane == i1, jnp.float32(-1.0), p)
            m2 = _allmax(p2)
            i2 = _allmin(jnp.where(p2 == m2, lane, _PAD))
            idxv = jnp.where(
                lane < _L, i1 * _L + lane,
                jnp.where(lane < 2 * _L, i2 * _L + (lane - _L), 0))
            g = _shuf(eo_v[...], idxv)
            wv = jnp.where(lane < _L, m1,
                           jnp.where(lane < 2 * _L, m2, jnp.float32(0.0)))
            prod = g * wv
            shifted = _shuf(prod, jnp.where(lane < _PAD - _L, lane + _L, 0))
            fsum = prod + shifted
            oval_v[...] = jnp.where(lane < _L, fsum, jnp.float32(0.0))
            oidx_v[...] = jnp.where(lane == 0, i1,
                                    jnp.where(lane == 1, i2, 0))
            pltpu.sync_copy(oval_v, oval_hbm.at[b])
            pltpu.sync_copy(oidx_v, oidx_hbm.at[b])

    return run(probs_pad, eo_pad)


def kernel(hidden_states, Wq, bq, Wk, bk, Wv, bv, W_ea, b_ea, Wg1, bg1,
           Wg2, bg2, We1, be1, We2, be2):
    hs = hidden_states
    pooled, swh = _attn(hs, hs.astype(jnp.bfloat16),
                        Wq.astype(jnp.bfloat16), bq.reshape(1, _D),
                        Wk.astype(jnp.bfloat16), bk.reshape(1, _D))
    probs_pad, eo = _head(
        pooled, swh, Wv, bv.reshape(1, _D), W_ea, b_ea.reshape(1, _E),
        Wg1, bg1.reshape(1, _D), Wg2, bg2.reshape(1, _E),
        We1, be1.reshape(_E, 1, _D), We2, be2.reshape(_E, 1, _L))
    eo_pad = jnp.pad(
        jnp.transpose(eo, (1, 0, 2)).reshape(_B, _E * _L),
        ((0, 0), (0, _PAD - _E * _L)))
    vals, idx = _route_sc(probs_pad, eo_pad)
    return vals[:, :_L], probs_pad[:, :_E], idx[:, :_TOPK]


# speedup vs baseline: 1.0601x; 1.0601x over previous
import functools
import math

import jax
import jax.numpy as jnp
from jax import lax
from jax.experimental import pallas as pl
from jax.experimental.pallas import tpu as pltpu
from jax.experimental.pallas import tpu_sc as plsc

_B, _S, _D = 2, 2048, 768
_E, _L, _TOPK = 5, 2, 2
_RK = 512
_RA = 256
_PAD = 16


def _gelu(x):
    return 0.5 * x * (1.0 + lax.erf(x * (1.0 / math.sqrt(2.0))))


def _attn_body(hid_ref, wq_ref, bq_ref, wk_ref, bk_ref,
               pooled_ref, swh_ref, k_scr, cs_scr):
    i = pl.program_id(1)
    nblk = _S // _RA

    @pl.when(i == 0)
    def _init():
        k_scr[...] = (
            jnp.dot(hid_ref[0], wk_ref[...],
                    preferred_element_type=jnp.float32)
            + bk_ref[...]
        )
        pooled_ref[0] = jnp.sum(hid_ref[0], axis=0, keepdims=True) * (1.0 / _S)
        cs_scr[...] = jnp.zeros((1, _S), jnp.float32)

    @pl.when(i > 0)
    def _block():
        r0 = (i - 1) * _RA
        q = (
            jnp.dot(hid_ref[0, pl.ds(r0, _RA), :], wq_ref[...],
                    preferred_element_type=jnp.float32)
            + bq_ref[...]
        )
        p = jnp.exp(lax.dot_general(
            q, k_scr[...], (((1,), (1,)), ((), ())),
            preferred_element_type=jnp.float32,
        ) * (1.0 / math.sqrt(_D)))
        rw = (1.0 / _S) / jnp.sum(p, axis=1, keepdims=True)
        cs_scr[...] += lax.dot_general(
            rw, p, (((0,), (0,)), ((), ())),
            preferred_element_type=jnp.float32)

    @pl.when(i == nblk)
    def _fin():
        swh_ref[0] = jnp.dot(cs_scr[...], hid_ref[0],
                             preferred_element_type=jnp.float32)


def _attn(hs, Wq, bq2, Wk, bk2):
    return pl.pallas_call(
        _attn_body,
        grid=(_B, 1 + _S // _RA),
        in_specs=[
            pl.BlockSpec((1, _S, _D), lambda b, i: (b, 0, 0)),
            pl.BlockSpec((_D, _D), lambda b, i: (0, 0)),
            pl.BlockSpec((1, _D), lambda b, i: (0, 0)),
            pl.BlockSpec((_D, _D), lambda b, i: (0, 0)),
            pl.BlockSpec((1, _D), lambda b, i: (0, 0)),
        ],
        out_specs=[
            pl.BlockSpec((1, 1, _D), lambda b, i: (b, 0, 0)),
            pl.BlockSpec((1, 1, _D), lambda b, i: (b, 0, 0)),
        ],
        out_shape=[
            jax.ShapeDtypeStruct((_B, 1, _D), jnp.float32),
            jax.ShapeDtypeStruct((_B, 1, _D), jnp.float32),
        ],
        scratch_shapes=[pltpu.VMEM((_S, _D), jnp.float32),
                        pltpu.VMEM((1, _S), jnp.float32)],
        compiler_params=pltpu.CompilerParams(
            dimension_semantics=("arbitrary", "arbitrary")),
    )(hs, Wq, bq2, Wk, bk2)


def _head_body(pooled_ref, swh_ref, wv_ref, bv_ref, wea_ref, bea_ref,
               wg1_ref, bg1_ref, wg2_ref, bg2_ref,
               we1_ref, be1_ref, we2_ref, be2_ref,
               probs_ref, eo_ref):
    e = pl.program_id(0)

    @pl.when(e == 0)
    def _gate():
        seq = (
            jnp.dot(swh_ref[:, 0, :], wv_ref[...],
                    preferred_element_type=jnp.float32)
            + bv_ref[...]
        )
        ea = jnp.dot(seq, wea_ref[...],
                     preferred_element_type=jnp.float32) + bea_ref[...]
        g1 = _gelu(jnp.dot(seq, wg1_ref[...],
                           preferred_element_type=jnp.float32) + bg1_ref[...])
        logits = ea + jnp.dot(g1, wg2_ref[...],
                              preferred_element_type=jnp.float32) + bg2_ref[...]
        mx = jnp.max(logits, axis=1, keepdims=True)
        ex = jnp.exp(logits - mx)
        probs = ex / jnp.sum(ex, axis=1, keepdims=True)
        probs_ref[:, 0:_E] = probs
        probs_ref[:, _E:] = jnp.zeros((_B, _PAD - _E), jnp.float32)

    h1 = _gelu(
        jnp.dot(pooled_ref[:, 0, :], we1_ref[0],
                preferred_element_type=jnp.float32)
        + be1_ref[0]
    )
    eo_ref[0] = jnp.dot(h1, we2_ref[0],
                        preferred_element_type=jnp.float32) + be2_ref[0]


def _head(pooled, swh, Wv, bv2, W_ea, bea2, Wg1, bg12, Wg2, bg22,
          We1, be1_3, We2, be2_3):
    return pl.pallas_call(
        _head_body,
        grid=(_E,),
        in_specs=[
            pl.BlockSpec((_B, 1, _D), lambda e: (0, 0, 0)),
            pl.BlockSpec((_B, 1, _D), lambda e: (0, 0, 0)),
            pl.BlockSpec((_D, _D), lambda e: (0, 0)),
            pl.BlockSpec((1, _D), lambda e: (0, 0)),
            pl.BlockSpec((_D, _E), lambda e: (0, 0)),
            pl.BlockSpec((1, _E), lambda e: (0, 0)),
            pl.BlockSpec((_D, _D), lambda e: (0, 0)),
            pl.BlockSpec((1, _D), lambda e: (0, 0)),
            pl.BlockSpec((_D, _E), lambda e: (0, 0)),
            pl.BlockSpec((1, _E), lambda e: (0, 0)),
            pl.BlockSpec((1, _D, _D), lambda e: (e, 0, 0)),
            pl.BlockSpec((1, 1, _D), lambda e: (e, 0, 0)),
            pl.BlockSpec((1, _D, _L), lambda e: (e, 0, 0)),
            pl.BlockSpec((1, 1, _L), lambda e: (e, 0, 0)),
        ],
        out_specs=[
            pl.BlockSpec((_B, _PAD), lambda e: (0, 0)),
            pl.BlockSpec((1, _B, _L), lambda e: (e, 0, 0)),
        ],
        out_shape=[
            jax.ShapeDtypeStruct((_B, _PAD), jnp.float32),
            jax.ShapeDtypeStruct((_E, _B, _L), jnp.float32),
        ],
        compiler_params=pltpu.CompilerParams(
            dimension_semantics=("arbitrary",)),
    )(pooled, swh, Wv, bv2, W_ea, bea2, Wg1, bg12, Wg2, bg22,
      We1, be1_3, We2, be2_3)


def _route_sc(probs_pad, eo_pad):
    mesh = plsc.VectorSubcoreMesh(core_axis_name="c", subcore_axis_name="s")

    @functools.partial(
        pl.kernel,
        mesh=mesh,
        out_type=[
            jax.ShapeDtypeStruct((_B, _PAD), jnp.float32),
            jax.ShapeDtypeStruct((_B, _PAD), jnp.int32),
        ],
        scratch_types=[
            pltpu.VMEM((_PAD,), jnp.float32),
            pltpu.VMEM((_PAD,), jnp.float32),
            pltpu.VMEM((_PAD,), jnp.float32),
            pltpu.VMEM((_PAD,), jnp.int32),
        ],
    )
    def run(probs_hbm, eo_hbm, oval_hbm, oidx_hbm,
            probs_v, eo_v, oval_v, oidx_v):
        wid = lax.axis_index("s") * 2 + lax.axis_index("c")

        @pl.when(wid < _B)
        def _():
            b = wid
            pltpu.sync_copy(probs_hbm.at[b], probs_v)
            pltpu.sync_copy(eo_hbm.at[b], eo_v)
            p = probs_v[...]
            lane = lax.iota(jnp.int32, _PAD)

            def _shuf(x, i):
                dnums = lax.GatherDimensionNumbers(
                    offset_dims=(), collapsed_slice_dims=(0,),
                    start_index_map=(0,))
                return lax.gather(
                    x, i[:, None], dnums, (1,),
                    mode=lax.GatherScatterMode.PROMISE_IN_BOUNDS)

            def _allmax(v):
                for sh in (1, 2, 4, 8):
                    v = jnp.maximum(v, _shuf(v, lane ^ sh))
                return v

            def _allmin(v):
                for sh in (1, 2, 4, 8):
                    v = jnp.minimum(v, _shuf(v, lane ^ sh))
                return v

            m1 = _allmax(p)
            i1 = _allmin(jnp.where(p == m1, lane, _PAD))
            p2 = jnp.where(lane == i1, jnp.float32(-1.0), p)
            m2 = _allmax(p2)
            i2 = _allmin(jnp.where(p2 == m2, lane, _PAD))
            idxv = jnp.where(
                lane < _L, i1 * _L + lane,
                jnp.where(lane < 2 * _L, i2 * _L + (lane - _L), 0))
            g = _shuf(eo_v[...], idxv)
            wv = jnp.where(lane < _L, m1,
                           jnp.where(lane < 2 * _L, m2, jnp.float32(0.0)))
            prod = g * wv
            shifted = _shuf(prod, jnp.where(lane < _PAD - _L, lane + _L, 0))
            fsum = prod + shifted
            oval_v[...] = jnp.where(lane < _L, fsum, jnp.float32(0.0))
            oidx_v[...] = jnp.where(lane == 0, i1,
                                    jnp.where(lane == 1, i2, 0))
            pltpu.sync_copy(oval_v, oval_hbm.at[b])
            pltpu.sync_copy(oidx_v, oidx_hbm.at[b])

    return run(probs_pad, eo_pad)


def kernel(hidden_states, Wq, bq, Wk, bk, Wv, bv, W_ea, b_ea, Wg1, bg1,
           Wg2, bg2, We1, be1, We2, be2):
    hs = hidden_states
    pooled, swh = _attn(hs, Wq, bq.reshape(1, _D), Wk, bk.reshape(1, _D))
    probs_pad, eo = _head(
        pooled, swh, Wv, bv.reshape(1, _D), W_ea, b_ea.reshape(1, _E),
        Wg1, bg1.reshape(1, _D), Wg2, bg2.reshape(1, _E),
        We1, be1.reshape(_E, 1, _D), We2, be2.reshape(_E, 1, _L))
    eo_pad = jnp.pad(
        jnp.transpose(eo, (1, 0, 2)).reshape(_B, _E * _L),
        ((0, 0), (0, _PAD - _E * _L)))
    vals, idx = _route_sc(probs_pad, eo_pad)
    return vals[:, :_L], probs_pad[:, :_E], idx[:, :_TOPK]


# speedup vs baseline: 1.2526x; 1.1815x over previous
import functools
import math

import jax
import jax.numpy as jnp
from jax import lax
from jax.experimental import pallas as pl
from jax.experimental.pallas import tpu as pltpu
from jax.experimental.pallas import tpu_sc as plsc

_B, _S, _D = 2, 2048, 768
_E, _L, _TOPK = 5, 2, 2
_RA = 256
_PAD = 16


def _gelu(x):
    return 0.5 * x * (1.0 + lax.erf(x * (1.0 / math.sqrt(2.0))))


def _attn_body(hid_ref, wq_ref, wk_ref, bq_ref,
               pooled_ref, swh_ref, a_scr, p_scr, cv_scr, cs_scr):
    b = pl.program_id(0)
    i = pl.program_id(1)
    nblk = _S // _RA
    scale = 1.0 / math.sqrt(_D)

    @pl.when((b == 0) & (i == 0))
    def _amat():
        a_scr[...] = lax.dot_general(
            wq_ref[...], wk_ref[...], (((1,), (1,)), ((), ())),
            preferred_element_type=jnp.float32) * scale

    @pl.when(i == 0)
    def _init():
        hid = hid_ref[0]
        p_scr[...] = jnp.dot(hid, a_scr[...],
                             preferred_element_type=jnp.float32)
        wkbq = lax.dot_general(
            jnp.reshape(bq_ref[...], (1, _D)), wk_ref[...],
            (((1,), (1,)), ((), ())),
            preferred_element_type=jnp.float32) * scale
        cv_scr[...] = lax.dot_general(
            wkbq, hid, (((1,), (1,)), ((), ())),
            preferred_element_type=jnp.float32)
        pooled_ref[0] = jnp.sum(hid, axis=0, keepdims=True) * (1.0 / _S)
        cs_scr[...] = jnp.zeros((1, _S), jnp.float32)

    @pl.when(i > 0)
    def _block():
        r0 = (i - 1) * _RA
        p = jnp.exp(lax.dot_general(
            p_scr[pl.ds(r0, _RA), :], hid_ref[0], (((1,), (1,)), ((), ())),
            preferred_element_type=jnp.float32,
        ) + cv_scr[...])
        rw = (1.0 / _S) / jnp.sum(p, axis=1, keepdims=True)
        cs_scr[...] += lax.dot_general(
            rw, p, (((0,), (0,)), ((), ())),
            preferred_element_type=jnp.float32)

    @pl.when(i == nblk)
    def _fin():
        swh_ref[0] = jnp.dot(cs_scr[...], hid_ref[0],
                             preferred_element_type=jnp.float32)


def _attn(hs, Wq, Wk, bq):
    return pl.pallas_call(
        _attn_body,
        grid=(_B, 1 + _S // _RA),
        in_specs=[
            pl.BlockSpec((1, _S, _D), lambda b, i: (b, 0, 0)),
            pl.BlockSpec((_D, _D), lambda b, i: (0, 0)),
            pl.BlockSpec((_D, _D), lambda b, i: (0, 0)),
            pl.BlockSpec((_D,), lambda b, i: (0,)),
        ],
        out_specs=[
            pl.BlockSpec((1, 1, _D), lambda b, i: (b, 0, 0)),
            pl.BlockSpec((1, 1, _D), lambda b, i: (b, 0, 0)),
        ],
        out_shape=[
            jax.ShapeDtypeStruct((_B, 1, _D), jnp.float32),
            jax.ShapeDtypeStruct((_B, 1, _D), jnp.float32),
        ],
        scratch_shapes=[pltpu.VMEM((_D, _D), jnp.float32),
                        pltpu.VMEM((_S, _D), jnp.float32),
                        pltpu.VMEM((1, _S), jnp.float32),
                        pltpu.VMEM((1, _S), jnp.float32)],
        compiler_params=pltpu.CompilerParams(
            dimension_semantics=("arbitrary", "arbitrary")),
    )(hs, Wq, Wk, bq)


def _head_body(pooled_ref, swh_ref, wv_ref, bv_ref, wea_ref, bea_ref,
               wg1_ref, bg1_ref, wg2_ref, bg2_ref,
               we1_ref, be1_ref, we2_ref, be2_ref,
               probs_ref, eo_ref):
    seq = (
        jnp.dot(swh_ref[:, 0, :], wv_ref[...],
                preferred_element_type=jnp.float32)
        + jnp.reshape(bv_ref[...], (1, _D))
    )
    ea = (jnp.dot(seq, wea_ref[...], preferred_element_type=jnp.float32)
          + jnp.reshape(bea_ref[...], (1, _E)))
    g1 = _gelu(jnp.dot(seq, wg1_ref[...], preferred_element_type=jnp.float32)
               + jnp.reshape(bg1_ref[...], (1, _D)))
    logits = ea + jnp.dot(g1, wg2_ref[...],
                          preferred_element_type=jnp.float32) \
        + jnp.reshape(bg2_ref[...], (1, _E))
    mx = jnp.max(logits, axis=1, keepdims=True)
    ex = jnp.exp(logits - mx)
    probs = ex / jnp.sum(ex, axis=1, keepdims=True)
    probs_ref[:, 0:_E] = probs
    probs_ref[:, _E:] = jnp.zeros((_B, _PAD - _E), jnp.float32)

    pooled = pooled_ref[:, 0, :]
    eo_ref[:, _E * _L:] = jnp.zeros((_B, _PAD - _E * _L), jnp.float32)
    for e in range(_E):
        h1 = _gelu(jnp.dot(pooled, we1_ref[e],
                           preferred_element_type=jnp.float32)
                   + jnp.reshape(be1_ref[e], (1, _D)))
        eo_ref[:, e * _L:(e + 1) * _L] = (
            jnp.dot(h1, we2_ref[e], preferred_element_type=jnp.float32)
            + jnp.reshape(be2_ref[e], (1, _L)))


def _head(pooled, swh, Wv, bv, W_ea, b_ea, Wg1, bg1, Wg2, bg2,
          We1, be1, We2, be2):
    return pl.pallas_call(
        _head_body,
        out_shape=[
            jax.ShapeDtypeStruct((_B, _PAD), jnp.float32),
            jax.ShapeDtypeStruct((_B, _PAD), jnp.float32),
        ],
    )(pooled, swh, Wv, bv, W_ea, b_ea, Wg1, bg1, Wg2, bg2,
      We1, be1, We2, be2)


def _route_sc(probs_pad, eo_pad):
    mesh = plsc.VectorSubcoreMesh(core_axis_name="c", subcore_axis_name="s")

    @functools.partial(
        pl.kernel,
        mesh=mesh,
        out_type=[
            jax.ShapeDtypeStruct((_B, _PAD), jnp.float32),
            jax.ShapeDtypeStruct((_B, _PAD), jnp.int32),
        ],
        scratch_types=[
            pltpu.VMEM((_PAD,), jnp.float32),
            pltpu.VMEM((_PAD,), jnp.float32),
            pltpu.VMEM((_PAD,), jnp.float32),
            pltpu.VMEM((_PAD,), jnp.int32),
        ],
    )
    def run(probs_hbm, eo_hbm, oval_hbm, oidx_hbm,
            probs_v, eo_v, oval_v, oidx_v):
        wid = lax.axis_index("s") * 2 + lax.axis_index("c")

        @pl.when(wid < _B)
        def _():
            b = wid
            pltpu.sync_copy(probs_hbm.at[b], probs_v)
            pltpu.sync_copy(eo_hbm.at[b], eo_v)
            p = probs_v[...]
            lane = lax.iota(jnp.int32, _PAD)

            def _shuf(x, i):
                dnums = lax.GatherDimensionNumbers(
                    offset_dims=(), collapsed_slice_dims=(0,),
                    start_index_map=(0,))
                return lax.gather(
                    x, i[:, None], dnums, (1,),
                    mode=lax.GatherScatterMode.PROMISE_IN_BOUNDS)

            def _allmax(v):
                for sh in (1, 2, 4, 8):
                    v = jnp.maximum(v, _shuf(v, lane ^ sh))
                return v

            def _allmin(v):
                for sh in (1, 2, 4, 8):
                    v = jnp.minimum(v, _shuf(v, lane ^ sh))
                return v

            m1 = _allmax(p)
            i1 = _allmin(jnp.where(p == m1, lane, _PAD))
            p2 = jnp.where(lane == i1, jnp.float32(-1.0), p)
            m2 = _allmax(p2)
            i2 = _allmin(jnp.where(p2 == m2, lane, _PAD))
            idxv = jnp.where(
                lane < _L, i1 * _L + lane,
                jnp.where(lane < 2 * _L, i2 * _L + (lane - _L), 0))
            g = _shuf(eo_v[...], idxv)
            wv = jnp.where(lane < _L, m1,
                           jnp.where(lane < 2 * _L, m2, jnp.float32(0.0)))
            prod = g * wv
            shifted = _shuf(prod, jnp.where(lane < _PAD - _L, lane + _L, 0))
            fsum = prod + shifted
            oval_v[...] = jnp.where(lane < _L, fsum, jnp.float32(0.0))
            oidx_v[...] = jnp.where(lane == 0, i1,
                                    jnp.where(lane == 1, i2, 0))
            pltpu.sync_copy(oval_v, oval_hbm.at[b])
            pltpu.sync_copy(oidx_v, oidx_hbm.at[b])

    return run(probs_pad, eo_pad)


def kernel(hidden_states, Wq, bq, Wk, bk, Wv, bv, W_ea, b_ea, Wg1, bg1,
           Wg2, bg2, We1, be1, We2, be2):
    pooled, swh = _attn(hidden_states, Wq, Wk, bq)
    probs_pad, eo_pad = _head(
        pooled, swh, Wv, bv, W_ea, b_ea, Wg1, bg1, Wg2, bg2,
        We1, be1, We2, be2)
    vals, idx = _route_sc(probs_pad, eo_pad)
    return vals[:, :_L], probs_pad[:, :_E], idx[:, :_TOPK]
